# Initial kernel scaffold; baseline (speedup 1.0000x reference)
import jax
import jax.numpy as jnp
from jax.experimental import pallas as pl


def kernel(input, expert_sel, keys, values, route_scale):
    raise NotImplementedError("write your pallas kernel here")


# speedup vs baseline: 3.2988x; 3.2988x over previous
import functools

import jax
import jax.numpy as jnp
from jax.experimental import pallas as pl
from jax.experimental.pallas import tpu as pltpu

B, T, D = 2, 2048, 1024
E, H, K = 8, 512, 2
BT = B * T
BM = 256


def _moe_body(x_ref, selT_ref, k_ref, v_ref, rs_ref, o_ref):
    j = pl.program_id(1)
    x = x_ref[...]
    logits = jnp.dot(x, selT_ref[...], preferred_element_type=jnp.float32)
    p = jax.nn.sigmoid(logits)
    eidx = jax.lax.broadcasted_iota(jnp.int32, (BM, E), 1)
    cnt = jnp.zeros((BM, E), jnp.int32)
    for a in range(E):
        pa = p[:, a : a + 1]
        beats = (pa > p) | ((pa == p) & (a < eidx))
        cnt = cnt + beats.astype(jnp.int32)
    sel = cnt < K
    g = jnp.where(sel, p, 0.0)
    denom = jnp.sum(g, axis=1, keepdims=True)
    w = g / jnp.maximum(denom, 1e-9) * rs_ref[0]
    wj = jnp.sum(jnp.where(eidx == j, w, 0.0), axis=1, keepdims=True)

    h = jnp.maximum(jnp.dot(x, k_ref[0], preferred_element_type=jnp.float32), 0.0)
    o = jnp.dot(h, v_ref[0], preferred_element_type=jnp.float32)
    contrib = o * wj

    @pl.when(j == 0)
    def _():
        o_ref[...] = contrib

    @pl.when(j > 0)
    def _():
        o_ref[...] = o_ref[...] + contrib


@functools.partial(jax.jit, static_argnames=("interpret",))
def _moe(x2d, selT, keys, values, route_scale, interpret=False):
    grid = (BT // BM, E)
    out = pl.pallas_call(
        _moe_body,
        grid=grid,
        in_specs=[
            pl.BlockSpec((BM, D), lambda i, j: (i, 0)),
            pl.BlockSpec((D, E), lambda i, j: (0, 0)),
            pl.BlockSpec((1, D, H), lambda i, j: (j, 0, 0)),
            pl.BlockSpec((1, H, D), lambda i, j: (j, 0, 0)),
            pl.BlockSpec(memory_space=pltpu.SMEM),
        ],
        out_specs=pl.BlockSpec((BM, D), lambda i, j: (i, 0)),
        out_shape=jax.ShapeDtypeStruct((BT, D), jnp.float32),
        interpret=interpret,
    )(x2d, selT, keys, values, route_scale)
    return out


def kernel(input, expert_sel, keys, values, route_scale, interpret=False):
    x2d = input.reshape(BT, D)
    selT = expert_sel.T
    out = _moe(x2d, selT, keys, values, route_scale, interpret=interpret)
    return out.reshape(B, T, D)


# speedup vs baseline: 6.7336x; 2.0412x over previous
import functools

import jax
import jax.numpy as jnp
from jax.experimental import pallas as pl
from jax.experimental.pallas import tpu as pltpu

B, T, D = 2, 2048, 1024
E, H, K = 8, 512, 2
BT = B * T
BM = 256


def _moe_body(x_ref, selT_ref, k_ref, v_ref, rs_ref, o_ref):
    x = x_ref[...]
    logits = jnp.dot(x, selT_ref[...], preferred_element_type=jnp.float32)
    p = jax.nn.sigmoid(logits)
    eidx = jax.lax.broadcasted_iota(jnp.int32, (BM, E), 1)
    cnt = jnp.zeros((BM, E), jnp.int32)
    for a in range(E):
        pa = p[:, a : a + 1]
        beats = (pa > p) | ((pa == p) & (a < eidx))
        cnt = cnt + beats.astype(jnp.int32)
    sel = cnt < K
    g = jnp.where(sel, p, 0.0)
    denom = jnp.sum(g, axis=1, keepdims=True)
    w = g / jnp.maximum(denom, 1e-9) * rs_ref[0]

    xb = x.astype(jnp.bfloat16)
    acc = jnp.zeros((BM, D), jnp.float32)
    for j in range(E):
        h = jnp.dot(xb, k_ref[j], preferred_element_type=jnp.float32)
        hb = jnp.maximum(h, 0.0).astype(jnp.bfloat16)
        oj = jnp.dot(hb, v_ref[j], preferred_element_type=jnp.float32)
        wj = jnp.sum(jnp.where(eidx == j, w, 0.0), axis=1, keepdims=True)
        acc = acc + oj * wj
    o_ref[...] = acc


@functools.partial(jax.jit, static_argnames=("interpret",))
def _moe(x2d, selT, keys, values, route_scale, interpret=False):
    grid = (BT // BM,)
    out = pl.pallas_call(
        _moe_body,
        grid=grid,
        in_specs=[
            pl.BlockSpec((BM, D), lambda i: (i, 0)),
            pl.BlockSpec((D, E), lambda i: (0, 0)),
            pl.BlockSpec((E, D, H), lambda i: (0, 0, 0)),
            pl.BlockSpec((E, H, D), lambda i: (0, 0, 0)),
            pl.BlockSpec(memory_space=pltpu.SMEM),
        ],
        out_specs=pl.BlockSpec((BM, D), lambda i: (i, 0)),
        out_shape=jax.ShapeDtypeStruct((BT, D), jnp.float32),
        interpret=interpret,
    )(x2d, selT, keys, values, route_scale)
    return out


def kernel(input, expert_sel, keys, values, route_scale, interpret=False):
    x2d = input.reshape(BT, D)
    selT = expert_sel.T
    kb = keys.astype(jnp.bfloat16)
    vb = values.astype(jnp.bfloat16)
    out = _moe(x2d, selT, kb, vb, route_scale, interpret=interpret)
    return out.reshape(B, T, D)
